# Initial kernel scaffold; baseline (speedup 1.0000x reference)
import jax
import jax.numpy as jnp
from jax.experimental import pallas as pl


def kernel(points_homogeneous, covariance_3d, extrinsic_matrix, focal_x, focal_y, tan_fovX, tan_fovY):
    raise NotImplementedError("write your pallas kernel here")


# speedup vs baseline: 16.5691x; 16.5691x over previous
import jax
import jax.numpy as jnp
from jax.experimental import pallas as pl


_BLK = 16000


def _tc_body(consts_ref, ptsT_ref, covT_ref, out_ref):
    c = consts_ref
    E00, E10, E20, E30 = c[0, 0], c[0, 1], c[0, 2], c[0, 3]
    E01, E11, E21, E31 = c[0, 4], c[0, 5], c[0, 6], c[0, 7]
    E02, E12, E22, E32 = c[0, 8], c[0, 9], c[0, 10], c[0, 11]
    R00, R01, R02 = c[0, 12], c[0, 13], c[0, 14]
    R10, R11, R12 = c[0, 15], c[0, 16], c[0, 17]
    R20, R21, R22 = c[0, 18], c[0, 19], c[0, 20]
    sxx, sxy, syy = c[0, 21], c[0, 22], c[0, 23]

    def bf(t):
        return t.astype(jnp.bfloat16).astype(jnp.float32)

    x = bf(ptsT_ref[0:1, :])
    y = bf(ptsT_ref[1:2, :])
    z = bf(ptsT_ref[2:3, :])
    pcx = E00 * x + E10 * y + E20 * z + E30
    pcy = E01 * x + E11 * y + E21 * z + E31
    pcz = E02 * x + E12 * y + E22 * z + E32

    v00 = pcz * R00 + pcx * R02
    v01 = pcz * R10 + pcx * R12
    v02 = pcz * R20 + pcx * R22
    v10 = pcz * R01 + pcy * R02
    v11 = pcz * R11 + pcy * R12
    v12 = pcz * R21 + pcy * R22

    c00 = covT_ref[0:1, :]
    c01 = covT_ref[1:2, :]
    c02 = covT_ref[2:3, :]
    c10 = covT_ref[3:4, :]
    c11 = covT_ref[4:5, :]
    c12 = covT_ref[5:6, :]
    c20 = covT_ref[6:7, :]
    c21 = covT_ref[7:8, :]
    c22 = covT_ref[8:9, :]

    w00 = c00 * v00 + c01 * v01 + c02 * v02
    w01 = c10 * v00 + c11 * v01 + c12 * v02
    w02 = c20 * v00 + c21 * v01 + c22 * v02
    w10 = c00 * v10 + c01 * v11 + c02 * v12
    w11 = c10 * v10 + c11 * v11 + c12 * v12
    w12 = c20 * v10 + c21 * v11 + c22 * v12

    m00 = v00 * w00 + v01 * w01 + v02 * w02
    m01 = v00 * w10 + v01 * w11 + v02 * w12
    m10 = v10 * w00 + v11 * w01 + v12 * w02
    m11 = v10 * w10 + v11 * w11 + v12 * w12

    zinv = 1.0 / pcz
    zi2 = zinv * zinv
    zi4 = zi2 * zi2
    out_ref[0:1, :] = m00 * (sxx * zi4)
    out_ref[1:2, :] = m01 * (sxy * zi4)
    out_ref[2:3, :] = m10 * (sxy * zi4)
    out_ref[3:4, :] = m11 * (syy * zi4)


def kernel(points_homogeneous, covariance_3d, extrinsic_matrix,
           focal_x, focal_y, tan_fovX, tan_fovY):
    n = points_homogeneous.shape[0]
    E = extrinsic_matrix
    fx = jnp.float32(focal_x)
    fy = jnp.float32(focal_y)
    Eb = E.astype(jnp.bfloat16).astype(jnp.float32)
    consts = jnp.zeros((1, 128), jnp.float32)
    consts = consts.at[0, 0:4].set(Eb[:, 0])
    consts = consts.at[0, 4:8].set(Eb[:, 1])
    consts = consts.at[0, 8:12].set(Eb[:, 2])
    consts = consts.at[0, 12:15].set(E[0, :3])
    consts = consts.at[0, 15:18].set(E[1, :3])
    consts = consts.at[0, 18:21].set(E[2, :3])
    consts = consts.at[0, 21].set(fx * fx)
    consts = consts.at[0, 22].set(fx * fy)
    consts = consts.at[0, 23].set(fy * fy)

    ptsT = points_homogeneous.T
    covT = covariance_3d.reshape(n, 9).T

    blk = _BLK if n % _BLK == 0 else n
    grid = n // blk
    outT = pl.pallas_call(
        _tc_body,
        grid=(grid,),
        in_specs=[
            pl.BlockSpec((1, 128), lambda i: (0, 0)),
            pl.BlockSpec((4, blk), lambda i: (0, i)),
            pl.BlockSpec((9, blk), lambda i: (0, i)),
        ],
        out_specs=pl.BlockSpec((4, blk), lambda i: (0, i)),
        out_shape=jax.ShapeDtypeStruct((4, n), jnp.float32),
    )(consts, ptsT, covT)
    return outT.T.reshape(n, 2, 2)
